# Initial kernel scaffold; baseline (speedup 1.0000x reference)
import functools

import jax
import jax.numpy as jnp
from jax import lax
from jax.experimental import pallas as pl
from jax.experimental.pallas import tpu as pltpu
from jax.experimental.pallas import tpu_sc as plsc

D = 120
D3 = D * D * D
C = 11
RES = 1.0
NC = 2
NS = 16
L = 16
CHUNK = 192
HALF = CHUNK // 2
NBLK = CHUNK // L
A_PAD = NS * CHUNK


def _sc_body(coords_hbm, na_hbm, vol_hbm, out_hbm,
             cx, cy, cz, idx_v, gat_v, na_v, sem_in, sem_g, sem_out):
    cid = lax.axis_index("c")
    sid = lax.axis_index("s")
    wid = sid * NC + cid
    b = wid // NS
    sub = wid % NS
    a0 = sub * CHUNK

    cps = [pltpu.async_copy(coords_hbm.at[b, 0, pl.ds(a0, CHUNK)], cx, sem_in),
           pltpu.async_copy(coords_hbm.at[b, 1, pl.ds(a0, CHUNK)], cy, sem_in),
           pltpu.async_copy(coords_hbm.at[b, 2, pl.ds(a0, CHUNK)], cz, sem_in),
           pltpu.async_copy(na_hbm.at[b], na_v, sem_in)]
    for cp in cps:
        cp.wait()

    base_b = b * (C * D3)
    for k in range(NBLK):
        sl = pl.ds(k * L, L)
        xi = jnp.clip(cx[sl].astype(jnp.int32), 0, D - 1)
        yi = jnp.clip(cy[sl].astype(jnp.int32), 0, D - 1)
        zi = jnp.clip(cz[sl].astype(jnp.int32), 0, D - 1)
        lin = (xi * D + yi) * D + zi + base_b
        h, off = (k * L) // HALF, (k * L) % HALF
        for c in range(C):
            idx_v[c, h, pl.ds(off, L)] = lin + c * D3

    gathers = [pltpu.async_copy(vol_hbm.at[idx_v.at[c, h]], gat_v.at[c, h], sem_g)
               for c in range(C) for h in range(2)]
    for cp in gathers:
        cp.wait()

    na = na_v[:]
    outs = []
    for c in range(C):
        for k in range(NBLK):
            h, off = (k * L) // HALF, (k * L) % HALF
            aid = a0 + k * L + lax.iota(jnp.int32, L)
            v = gat_v[c, h, pl.ds(off, L)]
            gat_v[c, h, pl.ds(off, L)] = jnp.where(aid < na, v, 0.0)
        outs.append(pltpu.async_copy(gat_v.at[c], out_hbm.at[b, c, sub], sem_out))
    for cp in outs:
        cp.wait()


def kernel(volume, coords, num_atoms):
    B = volume.shape[0]
    A = coords.shape[1] // 3
    ct = coords.reshape(B, A, 3).transpose(0, 2, 1)
    ct = jnp.pad(ct, ((0, 0), (0, 0), (0, A_PAD - A)))
    na16 = jnp.broadcast_to(num_atoms[:, None], (B, L)).astype(jnp.int32)
    vol_flat = volume.reshape(-1)

    mesh = plsc.VectorSubcoreMesh(core_axis_name="c", subcore_axis_name="s")
    run = pl.kernel(
        _sc_body,
        out_type=jax.ShapeDtypeStruct((B, C, NS, 2, HALF), jnp.float32),
        mesh=mesh,
        scratch_types=[
            pltpu.VMEM((CHUNK,), jnp.float32),
            pltpu.VMEM((CHUNK,), jnp.float32),
            pltpu.VMEM((CHUNK,), jnp.float32),
            pltpu.VMEM((C, 2, HALF), jnp.int32),
            pltpu.VMEM((C, 2, HALF), jnp.float32),
            pltpu.VMEM((L,), jnp.int32),
            pltpu.SemaphoreType.DMA,
            pltpu.SemaphoreType.DMA,
            pltpu.SemaphoreType.DMA,
        ],
    )
    out = run(ct, na16, vol_flat)
    return out.reshape(B, C, A_PAD)[:, :, :A]


# speedup vs baseline: 16.8256x; 16.8256x over previous
import functools

import jax
import jax.numpy as jnp
from jax import lax
from jax.experimental import pallas as pl
from jax.experimental.pallas import tpu as pltpu
from jax.experimental.pallas import tpu_sc as plsc

D = 120
D3 = D * D * D
C = 11
RES = 1.0
NC = 2
NS = 16
L = 16
CHUNK = 192
HALF = CHUNK // 2
NBLK = CHUNK // L
A_PAD = NS * CHUNK


def _sc_body(coords_hbm, na_hbm, vol_hbm, out_hbm,
             cx, cy, cz, idx_v, gat_v, na_v, sem_in, sem_g, sem_out):
    cid = lax.axis_index("c")
    sid = lax.axis_index("s")
    wid = sid * NC + cid
    b = wid // NS
    sub = wid % NS
    a0 = sub * CHUNK

    cps = [pltpu.async_copy(coords_hbm.at[b, 0, pl.ds(a0, CHUNK)], cx, sem_in),
           pltpu.async_copy(coords_hbm.at[b, 1, pl.ds(a0, CHUNK)], cy, sem_in),
           pltpu.async_copy(coords_hbm.at[b, 2, pl.ds(a0, CHUNK)], cz, sem_in),
           pltpu.async_copy(na_hbm.at[b], na_v, sem_in)]
    for cp in cps:
        cp.wait()

    base_b = b * (C * D3)
    for k in range(NBLK):
        sl = pl.ds(k * L, L)
        xi = jnp.clip(cx[sl].astype(jnp.int32), 0, D - 1)
        yi = jnp.clip(cy[sl].astype(jnp.int32), 0, D - 1)
        zi = jnp.clip(cz[sl].astype(jnp.int32), 0, D - 1)
        lin = (xi * D + yi) * D + zi + base_b
        h, off = (k * L) // HALF, (k * L) % HALF
        for c in range(C):
            idx_v[c, h, pl.ds(off, L)] = lin + c * D3

    gathers = [pltpu.async_copy(vol_hbm.at[idx_v.at[c, h]], gat_v.at[c, h], sem_g)
               for c in range(C) for h in range(2)]
    for cp in gathers:
        cp.wait()

    na = na_v[:]
    outs = []
    for c in range(C):
        for k in range(NBLK):
            h, off = (k * L) // HALF, (k * L) % HALF
            aid = a0 + k * L + lax.iota(jnp.int32, L)
            v = gat_v[c, h, pl.ds(off, L)]
            gat_v[c, h, pl.ds(off, L)] = jnp.where(aid < na, v, 0.0)
        outs.append(pltpu.async_copy(gat_v.at[c], out_hbm.at[b, c, sub], sem_out))
    for cp in outs:
        cp.wait()


def kernel(volume, coords, num_atoms):
    B = volume.shape[0]
    A = coords.shape[1] // 3
    ct = coords.reshape(B, A, 3).transpose(0, 2, 1)
    ct = jnp.pad(ct, ((0, 0), (0, 0), (0, A_PAD - A)))
    na16 = jnp.broadcast_to(num_atoms[:, None], (B, L)).astype(jnp.int32)
    vol_flat = volume.reshape(-1)

    mesh = plsc.VectorSubcoreMesh(core_axis_name="c", subcore_axis_name="s")
    run = pl.kernel(
        _sc_body,
        out_type=jax.ShapeDtypeStruct((B, C, NS, 2, HALF), jnp.float32),
        mesh=mesh,
        compiler_params=pltpu.CompilerParams(use_tc_tiling_on_sc=False),
        scratch_types=[
            pltpu.VMEM((CHUNK,), jnp.float32),
            pltpu.VMEM((CHUNK,), jnp.float32),
            pltpu.VMEM((CHUNK,), jnp.float32),
            pltpu.VMEM((C, 2, HALF), jnp.int32),
            pltpu.VMEM((C, 2, HALF), jnp.float32),
            pltpu.VMEM((L,), jnp.int32),
            pltpu.SemaphoreType.DMA,
            pltpu.SemaphoreType.DMA,
            pltpu.SemaphoreType.DMA,
        ],
    )
    out = run(ct, na16, vol_flat)
    return out.reshape(B, C, A_PAD)[:, :, :A]


# speedup vs baseline: 30.8836x; 1.8355x over previous
import functools

import jax
import jax.numpy as jnp
from jax import lax
from jax.experimental import pallas as pl
from jax.experimental.pallas import tpu as pltpu
from jax.experimental.pallas import tpu_sc as plsc

D = 120
DPAD = 128
C = 11
NC = 2
NS = 16
L = 16
CHUNK = 192
HALF = CHUNK // 2
NBLK = CHUNK // L
A_PAD = NS * CHUNK


def _sc_body(coords_hbm, na_hbm, vol_hbm, out_hbm,
             cv, idx_v, gat_v, na_v, sem_in, sem_g, sem_out):
    cid = lax.axis_index("c")
    sid = lax.axis_index("s")
    wid = sid * NC + cid
    b = wid // NS
    sub = wid % NS
    a0 = sub * CHUNK

    cps = [pltpu.async_copy(coords_hbm.at[wid], cv, sem_in),
           pltpu.async_copy(na_hbm.at[b], na_v, sem_in)]
    for cp in cps:
        cp.wait()

    base_b = b * (C * D * D)
    for k in range(NBLK):
        sl = pl.ds(k * L, L)
        xi = jnp.clip(cv[0, sl].astype(jnp.int32), 0, D - 1)
        yi = jnp.clip(cv[1, sl].astype(jnp.int32), 0, D - 1)
        zi = jnp.clip(cv[2, sl].astype(jnp.int32), 0, D - 1)
        p0 = (xi * D + yi + base_b) * DPAD + zi
        h, off = (k * L) // HALF, (k * L) % HALF
        for c in range(C):
            idx_v[c, h, pl.ds(off, L)] = p0 + c * (D * D * DPAD)

    gathers = [pltpu.async_copy(vol_hbm.at[idx_v.at[c, h]], gat_v.at[c, h], sem_g)
               for c in range(C) for h in range(2)]
    for cp in gathers:
        cp.wait()

    na = na_v[:]
    outs = []
    for c in range(C):
        for k in range(NBLK):
            h, off = (k * L) // HALF, (k * L) % HALF
            aid = a0 + k * L + lax.iota(jnp.int32, L)
            v = gat_v[c, h, pl.ds(off, L)]
            gat_v[c, h, pl.ds(off, L)] = jnp.where(aid < na, v, 0.0)
        outs.append(pltpu.async_copy(gat_v.at[c], out_hbm.at[b, c, sub], sem_out))
    for cp in outs:
        cp.wait()


def kernel(volume, coords, num_atoms):
    B = volume.shape[0]
    A = coords.shape[1] // 3
    ct = coords.reshape(B, A, 3).transpose(0, 2, 1)
    ct = jnp.pad(ct, ((0, 0), (0, 0), (0, A_PAD - A)))
    carr = ct.reshape(B, 3, NS, CHUNK).transpose(0, 2, 1, 3).reshape(B * NS, 3, CHUNK)
    na16 = jnp.broadcast_to(num_atoms[:, None], (B, L)).astype(jnp.int32)
    volp = jnp.pad(volume.reshape(B * C * D, D, D), ((0, 0), (0, 0), (0, DPAD - D)))
    vol_flat = volp.reshape(-1)

    mesh = plsc.VectorSubcoreMesh(core_axis_name="c", subcore_axis_name="s")
    run = pl.kernel(
        _sc_body,
        out_type=jax.ShapeDtypeStruct((B, C, NS, 2, HALF), jnp.float32),
        mesh=mesh,
        compiler_params=pltpu.CompilerParams(use_tc_tiling_on_sc=False),
        scratch_types=[
            pltpu.VMEM((3, CHUNK), jnp.float32),
            pltpu.VMEM((C, 2, HALF), jnp.int32),
            pltpu.VMEM((C, 2, HALF), jnp.float32),
            pltpu.VMEM((L,), jnp.int32),
            pltpu.SemaphoreType.DMA,
            pltpu.SemaphoreType.DMA,
            pltpu.SemaphoreType.DMA,
        ],
    )
    out = run(carr, na16, vol_flat)
    return out.reshape(B, C, A_PAD)[:, :, :A]


# speedup vs baseline: 32.4683x; 1.0513x over previous
import functools

import jax
import jax.numpy as jnp
from jax import lax
from jax.experimental import pallas as pl
from jax.experimental.pallas import tpu as pltpu
from jax.experimental.pallas import tpu_sc as plsc

D = 120
DPAD = 128
C = 11
NC = 2
NS = 16
L = 16
A = 3000
CHUNK = 192
HALF = CHUNK // 2
NBLK = CHUNK // L
LAST0 = A - CHUNK


def _sc_body(coords_hbm, na_hbm, vol_hbm, out_hbm,
             cv, idx_v, gat_v, na_v, sem_in, sem_g, sem_out):
    cid = lax.axis_index("c")
    sid = lax.axis_index("s")
    wid = sid * NC + cid
    b = wid // NS
    sub = wid % NS
    a0 = jnp.where(sub == NS - 1, LAST0, sub * CHUNK)

    cps = [pltpu.async_copy(coords_hbm.at[b, pl.ds(a0 * 3, 3 * CHUNK)], cv, sem_in),
           pltpu.async_copy(na_hbm.at[b], na_v, sem_in)]
    for cp in cps:
        cp.wait()

    base_b = b * (C * D * D)
    for k in range(NBLK):
        a3 = (lax.iota(jnp.int32, L) + k * L) * 3
        xi = jnp.clip(plsc.load_gather(cv, [a3]).astype(jnp.int32), 0, D - 1)
        yi = jnp.clip(plsc.load_gather(cv, [a3 + 1]).astype(jnp.int32), 0, D - 1)
        zi = jnp.clip(plsc.load_gather(cv, [a3 + 2]).astype(jnp.int32), 0, D - 1)
        p0 = (xi * D + yi + base_b) * DPAD + zi
        h, off = (k * L) // HALF, (k * L) % HALF
        for c in range(C):
            idx_v[c, h, pl.ds(off, L)] = p0 + c * (D * D * DPAD)

    gathers = [pltpu.async_copy(vol_hbm.at[idx_v.at[c, h]],
                                gat_v.at[c, pl.ds(h * HALF, HALF)], sem_g)
               for c in range(C) for h in range(2)]
    for cp in gathers:
        cp.wait()

    na = na_v[:]
    outs = []
    for c in range(C):
        for k in range(NBLK):
            sl = pl.ds(k * L, L)
            aid = a0 + k * L + lax.iota(jnp.int32, L)
            gat_v[c, sl] = jnp.where(aid < na, gat_v[c, sl], 0.0)
        outs.append(pltpu.async_copy(gat_v.at[c], out_hbm.at[b, c, pl.ds(a0, CHUNK)],
                                     sem_out))
    for cp in outs:
        cp.wait()


def kernel(volume, coords, num_atoms):
    B = volume.shape[0]
    na16 = jnp.broadcast_to(num_atoms[:, None], (B, L)).astype(jnp.int32)
    volp = jnp.pad(volume.reshape(B * C * D, D, D), ((0, 0), (0, 0), (0, DPAD - D)))
    vol_flat = volp.reshape(-1)

    mesh = plsc.VectorSubcoreMesh(core_axis_name="c", subcore_axis_name="s")
    run = pl.kernel(
        _sc_body,
        out_type=jax.ShapeDtypeStruct((B, C, A), jnp.float32),
        mesh=mesh,
        compiler_params=pltpu.CompilerParams(use_tc_tiling_on_sc=False,
                                             needs_layout_passes=False),
        scratch_types=[
            pltpu.VMEM((3 * CHUNK,), jnp.float32),
            pltpu.VMEM((C, 2, HALF), jnp.int32),
            pltpu.VMEM((C, CHUNK), jnp.float32),
            pltpu.VMEM((L,), jnp.int32),
            pltpu.SemaphoreType.DMA,
            pltpu.SemaphoreType.DMA,
            pltpu.SemaphoreType.DMA,
        ],
    )
    return run(coords, na16, vol_flat)
